# baseline (speedup 1.0000x reference)
import functools

import jax
import jax.numpy as jnp
from jax import lax
from jax.experimental import pallas as pl
from jax.experimental.pallas import tpu as pltpu
from jax.experimental.pallas import tpu_sc as plsc

_CH = 8


def _sc_copy(nc, rows_per_w, n_chunks, x_hbm, o_hbm, buf):
    wid = lax.axis_index("s") * nc + lax.axis_index("c")
    base = wid * rows_per_w
    for c in range(n_chunks):
        off = base + c * _CH
        pltpu.sync_copy(x_hbm.at[pl.ds(off, _CH), :], buf)
        pltpu.sync_copy(buf, o_hbm.at[pl.ds(off, _CH), :])


def kernel(x):
    b, c, f = x.shape
    rows = b * c
    x2 = x.reshape(rows, f)

    info = plsc.get_sparse_core_info()
    nc, ns = info.num_cores, info.num_subcores
    nw = nc * ns
    rows_per_w = rows // nw
    n_chunks = rows_per_w // _CH

    mesh = plsc.VectorSubcoreMesh(core_axis_name="c", subcore_axis_name="s")
    out = pl.kernel(
        functools.partial(_sc_copy, nc, rows_per_w, n_chunks),
        out_type=jax.ShapeDtypeStruct((rows, f), x.dtype),
        mesh=mesh,
        scratch_types=[pltpu.VMEM((_CH, f), x.dtype)],
    )(x2)
    return out.reshape(b, c, f)


# speedup vs baseline: 1.1331x; 1.1331x over previous
import functools

import jax
import jax.numpy as jnp
from jax import lax
from jax.experimental import pallas as pl
from jax.experimental.pallas import tpu as pltpu
from jax.experimental.pallas import tpu_sc as plsc

_CH = 8


def _sc_copy(nc, rows_per_w, n_chunks, x_hbm, o_hbm, buf0, buf1, lsem, ssem):
    wid = lax.axis_index("s") * nc + lax.axis_index("c")
    base = wid * rows_per_w
    bufs = (buf0, buf1)
    lsems = (lsem.at[0], lsem.at[1])
    ssems = (ssem.at[0], ssem.at[1])

    def load(c):
        return pltpu.async_copy(
            x_hbm.at[pl.ds(base + c * _CH, _CH), :], bufs[c % 2], lsems[c % 2]
        )

    def store(c):
        return pltpu.async_copy(
            bufs[c % 2], o_hbm.at[pl.ds(base + c * _CH, _CH), :], ssems[c % 2]
        )

    load_h = {0: load(0)}
    store_h = {}
    for c in range(n_chunks):
        if c + 1 < n_chunks:
            if c >= 1:
                store_h[c - 1].wait()
            load_h[c + 1] = load(c + 1)
        load_h[c].wait()
        store_h[c] = store(c)
    store_h[n_chunks - 1].wait()


def kernel(x):
    b, c, f = x.shape
    rows = b * c
    x2 = x.reshape(rows, f)

    info = plsc.get_sparse_core_info()
    nc, ns = info.num_cores, info.num_subcores
    nw = nc * ns
    rows_per_w = rows // nw
    n_chunks = rows_per_w // _CH

    mesh = plsc.VectorSubcoreMesh(core_axis_name="c", subcore_axis_name="s")
    out = pl.kernel(
        functools.partial(_sc_copy, nc, rows_per_w, n_chunks),
        out_type=jax.ShapeDtypeStruct((rows, f), x.dtype),
        mesh=mesh,
        scratch_types=[
            pltpu.VMEM((_CH, f), x.dtype),
            pltpu.VMEM((_CH, f), x.dtype),
            pltpu.SemaphoreType.DMA((2,)),
            pltpu.SemaphoreType.DMA((2,)),
        ],
    )(x2)
    return out.reshape(b, c, f)


# speedup vs baseline: 1.1635x; 1.0269x over previous
import functools

import jax
import jax.numpy as jnp
from jax import lax
from jax.experimental import pallas as pl
from jax.experimental.pallas import tpu as pltpu
from jax.experimental.pallas import tpu_sc as plsc

_CH = 8
_K = 3
_L = 2


def _sc_copy(nc, rows_per_w, n_chunks, x_hbm, o_hbm, buf0, buf1, buf2, lsem, ssem):
    wid = lax.axis_index("s") * nc + lax.axis_index("c")
    base = wid * rows_per_w
    bufs = (buf0, buf1, buf2)

    def load(c):
        return pltpu.async_copy(
            x_hbm.at[pl.ds(base + c * _CH, _CH), :], bufs[c % _K], lsem.at[c % _K]
        )

    def store(c):
        return pltpu.async_copy(
            bufs[c % _K], o_hbm.at[pl.ds(base + c * _CH, _CH), :], ssem.at[c % _K]
        )

    load_h = {}
    store_h = {}
    for c in range(min(_L, n_chunks)):
        load_h[c] = load(c)
    for c in range(n_chunks):
        lc = c + _L
        if lc < n_chunks:
            pc = lc - _K
            if pc >= 0:
                store_h[pc].wait()
            load_h[lc] = load(lc)
        load_h[c].wait()
        store_h[c] = store(c)
    for c in range(max(0, n_chunks - _K), n_chunks):
        store_h[c].wait()


def kernel(x):
    b, c, f = x.shape
    rows = b * c
    x2 = x.reshape(rows, f)

    info = plsc.get_sparse_core_info()
    nc, ns = info.num_cores, info.num_subcores
    nw = nc * ns
    rows_per_w = rows // nw
    n_chunks = rows_per_w // _CH

    mesh = plsc.VectorSubcoreMesh(core_axis_name="c", subcore_axis_name="s")
    out = pl.kernel(
        functools.partial(_sc_copy, nc, rows_per_w, n_chunks),
        out_type=jax.ShapeDtypeStruct((rows, f), x.dtype),
        mesh=mesh,
        scratch_types=[
            pltpu.VMEM((_CH, f), x.dtype),
            pltpu.VMEM((_CH, f), x.dtype),
            pltpu.VMEM((_CH, f), x.dtype),
            pltpu.SemaphoreType.DMA((_K,)),
            pltpu.SemaphoreType.DMA((_K,)),
        ],
    )(x2)
    return out.reshape(b, c, f)


# speedup vs baseline: 1.8371x; 1.5789x over previous
import jax
import jax.numpy as jnp
from jax.experimental import pallas as pl
from jax.experimental.pallas import tpu as pltpu

_ROWS = 1016


def _copy_kernel(x_ref, o_ref):
    o_ref[...] = x_ref[...]


def kernel(x):
    b, c, f = x.shape
    x2 = x.reshape(b * c, f)
    out = pl.pallas_call(
        _copy_kernel,
        out_shape=jax.ShapeDtypeStruct(x2.shape, x2.dtype),
        grid=(pl.cdiv(b * c, _ROWS),),
        in_specs=[pl.BlockSpec((_ROWS, f), lambda i: (i, 0))],
        out_specs=pl.BlockSpec((_ROWS, f), lambda i: (i, 0)),
        compiler_params=pltpu.CompilerParams(vmem_limit_bytes=100 * 1024 * 1024),
    )(x2)
    return out.reshape(b, c, f)
